# baseline (speedup 1.0000x reference)
import functools

import jax
import jax.numpy as jnp
from jax import lax
from jax.experimental import pallas as pl
from jax.experimental.pallas import tpu as pltpu
from jax.experimental.pallas import tpu_sc as plsc

_N = 524288
_K = 8
_C = 256
_B = 32768
_NB = _N // _B
_SEG_BLOCKS = (5, 5, 5, 1)

_NC = 2
_NS = 16
_NW = _NC * _NS


def _score_argmax_body(xt_ref, aug_ref, idx_ref):
    aug = aug_ref[...]
    g2 = lax.slice(aug, (0, 0), (_C, _K))
    norm = lax.slice(aug, (0, _K), (_C, _K + 1))
    sc = jnp.dot(g2, xt_ref[...], preferred_element_type=jnp.float32)
    sc = sc - norm
    m = lax.slice(sc, (0, 0), (8, _B))
    tof = jnp.zeros((8, _B), jnp.int32)
    for t in range(1, _C // 8):
        v = lax.slice(sc, (t * 8, 0), (t * 8 + 8, _B))
        cond = v > m
        m = jnp.where(cond, v, m)
        tof = jnp.where(cond, t, tof)
    fm = jnp.max(m, axis=0, keepdims=True)
    j = tof * 8 + lax.broadcasted_iota(jnp.int32, (8, _B), 0)
    cand = jnp.where(m == fm, j, _C)
    idx_ref[0] = jnp.min(cand, axis=0).reshape(_B // 128, 128)


def _tc_score_argmax(xt, aug, base, nblocks):
    return pl.pallas_call(
        _score_argmax_body,
        grid=(nblocks,),
        in_specs=[
            pl.BlockSpec((_K, _B), lambda i: (0, base + i)),
            pl.BlockSpec((_C, 16), lambda i: (0, 0)),
        ],
        out_specs=pl.BlockSpec((1, _B // 128, 128), lambda i: (i, 0, 0)),
        out_shape=jax.ShapeDtypeStruct((nblocks, _B // 128, 128), jnp.int32),
    )(xt, aug)


def _sc_gather(idx_flat, table_t, out_cols):
    n = idx_flat.shape[0]
    rows_per_w = n // _NW
    chunk = min(4096, rows_per_w & (-rows_per_w))
    mesh = plsc.VectorSubcoreMesh(core_axis_name="c", subcore_axis_name="s")

    @functools.partial(
        pl.kernel,
        mesh=mesh,
        out_type=jax.ShapeDtypeStruct((_K, out_cols), jnp.float32),
        scratch_types=[
            pltpu.VMEM((rows_per_w,), jnp.int32),
            [pltpu.VMEM((_C,), jnp.float32) for _ in range(_K)],
            pltpu.VMEM((_K, chunk), jnp.float32),
            pltpu.SemaphoreType.DMA,
        ],
        compiler_params=pltpu.CompilerParams(needs_layout_passes=False),
    )
    def k(idx_hbm, table_hbm, out_hbm, idx_v, tabs, rows_v, sem):
        wid = lax.axis_index("s") * _NC + lax.axis_index("c")
        row_base = wid * rows_per_w
        copies = [pltpu.async_copy(table_hbm.at[kk], tabs[kk], sem)
                  for kk in range(_K)]
        copies.append(pltpu.async_copy(
            idx_hbm.at[pl.ds(pl.multiple_of(row_base, 128), rows_per_w)],
            idx_v, sem))
        for cp in copies:
            cp.wait()

        def do_chunk(c, carry):
            def body16(c2, carry2):
                coff = pl.multiple_of(c * chunk + c2 * 16, 8)
                rvec = idx_v[pl.ds(coff, 16)]
                soff = pl.multiple_of(c2 * 16, 8)
                for kk in range(_K):
                    rows_v[kk, pl.ds(soff, 16)] = plsc.load_gather(
                        tabs[kk], [rvec])
                return carry2

            lax.fori_loop(0, chunk // 16, body16, 0, unroll=8)
            ooff = pl.multiple_of(row_base + c * chunk, 128)
            pltpu.sync_copy(rows_v, out_hbm.at[:, pl.ds(ooff, chunk)])
            return carry

        lax.fori_loop(0, rows_per_w // chunk, do_chunk, 0)

    return k(idx_flat, table_t)


def kernel(X, grid, grid_norm):
    xt = X.T
    gt = grid.T
    aug = jnp.concatenate(
        [2.0 * grid, grid_norm.reshape(_C, 1),
         jnp.zeros((_C, 16 - _K - 1), jnp.float32)], axis=1)
    quantized = None
    idx_segs = []
    base = 0
    for seg, nblocks in enumerate(_SEG_BLOCKS):
        seg_rows = nblocks * _B
        idx3 = _tc_score_argmax(xt, aug, base, nblocks)
        idx_flat = idx3.reshape(seg_rows)
        if seg == 0:
            quantized = _sc_gather(idx_flat, gt, _N)
        else:
            qs = _sc_gather(idx_flat, gt, seg_rows)
            quantized = lax.dynamic_update_slice(
                quantized, qs, (0, base * _B))
        idx_segs.append(idx_flat.astype(jnp.uint8))
        base += nblocks
    return (quantized.T, jnp.concatenate(idx_segs))


# speedup vs baseline: 1.0071x; 1.0071x over previous
import functools

import jax
import jax.numpy as jnp
from jax import lax
from jax.experimental import pallas as pl
from jax.experimental.pallas import tpu as pltpu
from jax.experimental.pallas import tpu_sc as plsc

_N = 524288
_K = 8
_C = 256
_B = 16384
_NB = _N // _B
_SEG_BLOCKS = (10, 10, 10, 2)

_NC = 2
_NS = 16
_NW = _NC * _NS


def _score_argmax_body(xt_ref, aug_ref, idx_ref):
    aug = aug_ref[...]
    g2 = lax.slice(aug, (0, 0), (_C, _K))
    norm = lax.slice(aug, (0, _K), (_C, _K + 1))
    sc = jnp.dot(g2, xt_ref[...], preferred_element_type=jnp.float32)
    sc = sc - norm
    m = lax.slice(sc, (0, 0), (8, _B))
    tof = jnp.zeros((8, _B), jnp.int32)
    for t in range(1, _C // 8):
        v = lax.slice(sc, (t * 8, 0), (t * 8 + 8, _B))
        cond = v > m
        m = jnp.where(cond, v, m)
        tof = jnp.where(cond, t, tof)
    fm = jnp.max(m, axis=0, keepdims=True)
    j = tof * 8 + lax.broadcasted_iota(jnp.int32, (8, _B), 0)
    cand = jnp.where(m == fm, j, _C)
    idx_ref[0] = jnp.min(cand, axis=0).reshape(_B // 128, 128)


def _tc_score_argmax(xt, aug, base, nblocks):
    return pl.pallas_call(
        _score_argmax_body,
        grid=(nblocks,),
        in_specs=[
            pl.BlockSpec((_K, _B), lambda i: (0, base + i)),
            pl.BlockSpec((_C, 16), lambda i: (0, 0)),
        ],
        out_specs=pl.BlockSpec((1, _B // 128, 128), lambda i: (i, 0, 0)),
        out_shape=jax.ShapeDtypeStruct((nblocks, _B // 128, 128), jnp.int32),
    )(xt, aug)


def _sc_gather(idx_flat, table_t, out_cols):
    n = idx_flat.shape[0]
    rows_per_w = n // _NW
    chunk = min(4096, rows_per_w & (-rows_per_w))
    mesh = plsc.VectorSubcoreMesh(core_axis_name="c", subcore_axis_name="s")

    @functools.partial(
        pl.kernel,
        mesh=mesh,
        out_type=jax.ShapeDtypeStruct((_K, out_cols), jnp.float32),
        scratch_types=[
            pltpu.VMEM((rows_per_w,), jnp.int32),
            [pltpu.VMEM((_C,), jnp.float32) for _ in range(_K)],
            pltpu.VMEM((_K, chunk), jnp.float32),
            pltpu.SemaphoreType.DMA,
        ],
        compiler_params=pltpu.CompilerParams(needs_layout_passes=False),
    )
    def k(idx_hbm, table_hbm, out_hbm, idx_v, tabs, rows_v, sem):
        wid = lax.axis_index("s") * _NC + lax.axis_index("c")
        row_base = wid * rows_per_w
        copies = [pltpu.async_copy(table_hbm.at[kk], tabs[kk], sem)
                  for kk in range(_K)]
        copies.append(pltpu.async_copy(
            idx_hbm.at[pl.ds(pl.multiple_of(row_base, 128), rows_per_w)],
            idx_v, sem))
        for cp in copies:
            cp.wait()

        def do_chunk(c, carry):
            def body16(c2, carry2):
                coff = pl.multiple_of(c * chunk + c2 * 16, 8)
                rvec = idx_v[pl.ds(coff, 16)]
                soff = pl.multiple_of(c2 * 16, 8)
                for kk in range(_K):
                    rows_v[kk, pl.ds(soff, 16)] = plsc.load_gather(
                        tabs[kk], [rvec])
                return carry2

            lax.fori_loop(0, chunk // 16, body16, 0, unroll=8)
            ooff = pl.multiple_of(row_base + c * chunk, 128)
            pltpu.sync_copy(rows_v, out_hbm.at[:, pl.ds(ooff, chunk)])
            return carry

        lax.fori_loop(0, rows_per_w // chunk, do_chunk, 0)

    return k(idx_flat, table_t)


def kernel(X, grid, grid_norm):
    xt = X.T
    gt = grid.T
    aug = jnp.concatenate(
        [2.0 * grid, grid_norm.reshape(_C, 1),
         jnp.zeros((_C, 16 - _K - 1), jnp.float32)], axis=1)
    quantized = None
    idx_segs = []
    base = 0
    for seg, nblocks in enumerate(_SEG_BLOCKS):
        seg_rows = nblocks * _B
        idx3 = _tc_score_argmax(xt, aug, base, nblocks)
        idx_flat = idx3.reshape(seg_rows)
        if seg == 0:
            quantized = _sc_gather(idx_flat, gt, _N)
        else:
            qs = _sc_gather(idx_flat, gt, seg_rows)
            quantized = lax.dynamic_update_slice(
                quantized, qs, (0, base * _B))
        idx_segs.append(idx_flat.astype(jnp.uint8))
        base += nblocks
    return (quantized.T, jnp.concatenate(idx_segs))
